# baseline (speedup 1.0000x reference)
import functools

import jax
import jax.numpy as jnp
from jax import lax
from jax.experimental import pallas as pl
from jax.experimental.pallas import tpu as pltpu
from jax.experimental.pallas import tpu_sc as plsc

N = 1048576
NUM_CORES = 2
NUM_SUBCORES = 16
NUM_WORKERS = NUM_CORES * NUM_SUBCORES
PER_WORKER = N // NUM_WORKERS
CHUNK = 2048
NUM_CHUNKS = PER_WORKER // CHUNK
LANES = 16
NUM_SETS = 6


def _sort_chunk(in_bufs, out_bufs):
    def inner(i, c):
        for u in range(2):
            s = i * (2 * LANES) + u * LANES
            a = in_bufs[0][pl.ds(s, LANES)]
            b = in_bufs[1][pl.ds(s, LANES)]
            cc = in_bufs[2][pl.ds(s, LANES)]
            d = in_bufs[3][pl.ds(s, LANES)]
            lo1 = jnp.minimum(a, b)
            hi1 = jnp.maximum(a, b)
            lo2 = jnp.minimum(cc, d)
            hi2 = jnp.maximum(cc, d)
            o0 = jnp.minimum(lo1, lo2)
            o3 = jnp.maximum(hi1, hi2)
            m1 = jnp.maximum(lo1, lo2)
            m2 = jnp.minimum(hi1, hi2)
            out_bufs[0][pl.ds(s, LANES)] = o0
            out_bufs[1][pl.ds(s, LANES)] = jnp.minimum(m1, m2)
            out_bufs[2][pl.ds(s, LANES)] = jnp.maximum(m1, m2)
            out_bufs[3][pl.ds(s, LANES)] = o3
        return c

    lax.fori_loop(0, CHUNK // LANES // 2, inner, 0)


def _body(x1h, x2h, x3h, x4h, outh, *scratch):
    nb = NUM_SETS * 4
    in_bufs = tuple(tuple(scratch[st * 4 + j] for j in range(4))
                    for st in range(NUM_SETS))
    out_bufs = tuple(tuple(scratch[nb + st * 4 + j] for j in range(4))
                     for st in range(NUM_SETS))
    insems = scratch[2 * nb:2 * nb + NUM_SETS]
    outsems = scratch[2 * nb + NUM_SETS:]
    wid = lax.axis_index("s") * NUM_CORES + lax.axis_index("c")
    base = wid * PER_WORKER
    xs = (x1h, x2h, x3h, x4h)

    in_handles = [None] * NUM_SETS
    out_handles = [None] * NUM_SETS

    def start_inputs(ci):
        st = ci % NUM_SETS
        off = base + ci * CHUNK
        in_handles[st] = [
            pltpu.async_copy(xs[j].at[pl.ds(off, CHUNK)], in_bufs[st][j],
                             insems[st])
            for j in range(4)
        ]

    def start_outputs(ci):
        st = ci % NUM_SETS
        off = base + ci * CHUNK
        out_handles[st] = [
            pltpu.async_copy(out_bufs[st][j], outh.at[j, pl.ds(off, CHUNK)],
                             outsems[st])
            for j in range(4)
        ]

    for ci in range(min(NUM_SETS, NUM_CHUNKS)):
        start_inputs(ci)
    for ci in range(NUM_CHUNKS):
        st = ci % NUM_SETS
        for h in in_handles[st]:
            h.wait()
        if ci >= NUM_SETS:
            for h in out_handles[st]:
                h.wait()
        _sort_chunk(in_bufs[st], out_bufs[st])
        nxt = ci + NUM_SETS
        if nxt < NUM_CHUNKS:
            start_inputs(nxt)
        start_outputs(ci)
    for st in range(NUM_SETS):
        if out_handles[st] is not None:
            for h in out_handles[st]:
                h.wait()


def kernel(x1, x2, x3, x4):
    mesh = plsc.VectorSubcoreMesh(core_axis_name="c", subcore_axis_name="s")
    run = functools.partial(
        pl.kernel,
        mesh=mesh,
        out_type=jax.ShapeDtypeStruct((4, N), jnp.float32),
        scratch_types=(
            [pltpu.VMEM((CHUNK,), jnp.float32) for _ in range(NUM_SETS * 8)]
            + [pltpu.SemaphoreType.DMA for _ in range(NUM_SETS * 2)]
        ),
    )(_body)
    return run(x1, x2, x3, x4)


# speedup vs baseline: 1.0403x; 1.0403x over previous
import functools

import jax
import jax.numpy as jnp
from jax import lax
from jax.experimental import pallas as pl
from jax.experimental.pallas import tpu as pltpu
from jax.experimental.pallas import tpu_sc as plsc

N = 1048576
NUM_CORES = 2
NUM_SUBCORES = 16
NUM_WORKERS = NUM_CORES * NUM_SUBCORES
PER_WORKER = N // NUM_WORKERS
CHUNK = 4096
LANES = 16
NUM_SETS = 3

SIZES = (2048,) + (CHUNK,) * 7 + (2048,)
OFFSETS = tuple(sum(SIZES[:i]) for i in range(len(SIZES)))
NUM_CHUNKS = len(SIZES)
assert sum(SIZES) == PER_WORKER


def _sort_chunk(in_bufs, out_bufs, size):
    def inner(i, c):
        for u in range(2):
            s = i * (2 * LANES) + u * LANES
            a = in_bufs[0][pl.ds(s, LANES)]
            b = in_bufs[1][pl.ds(s, LANES)]
            cc = in_bufs[2][pl.ds(s, LANES)]
            d = in_bufs[3][pl.ds(s, LANES)]
            lo1 = jnp.minimum(a, b)
            hi1 = jnp.maximum(a, b)
            lo2 = jnp.minimum(cc, d)
            hi2 = jnp.maximum(cc, d)
            o0 = jnp.minimum(lo1, lo2)
            o3 = jnp.maximum(hi1, hi2)
            m1 = jnp.maximum(lo1, lo2)
            m2 = jnp.minimum(hi1, hi2)
            out_bufs[0][pl.ds(s, LANES)] = o0
            out_bufs[1][pl.ds(s, LANES)] = jnp.minimum(m1, m2)
            out_bufs[2][pl.ds(s, LANES)] = jnp.maximum(m1, m2)
            out_bufs[3][pl.ds(s, LANES)] = o3
        return c

    lax.fori_loop(0, size // LANES // 2, inner, 0)


def _body(x1h, x2h, x3h, x4h, outh, *scratch):
    nb = NUM_SETS * 4
    in_bufs = tuple(tuple(scratch[st * 4 + j] for j in range(4))
                    for st in range(NUM_SETS))
    out_bufs = tuple(tuple(scratch[nb + st * 4 + j] for j in range(4))
                     for st in range(NUM_SETS))
    insems = scratch[2 * nb:2 * nb + NUM_SETS]
    outsems = scratch[2 * nb + NUM_SETS:]
    wid = lax.axis_index("s") * NUM_CORES + lax.axis_index("c")
    base = wid * PER_WORKER
    xs = (x1h, x2h, x3h, x4h)

    in_handles = [None] * NUM_SETS
    out_handles = [None] * NUM_SETS

    def start_inputs(ci):
        st = ci % NUM_SETS
        off = base + OFFSETS[ci]
        sz = SIZES[ci]
        in_handles[st] = [
            pltpu.async_copy(xs[j].at[pl.ds(off, sz)],
                             in_bufs[st][j].at[pl.ds(0, sz)], insems[st])
            for j in range(4)
        ]

    def start_outputs(ci):
        st = ci % NUM_SETS
        off = base + OFFSETS[ci]
        sz = SIZES[ci]
        out_handles[st] = [
            pltpu.async_copy(out_bufs[st][j].at[pl.ds(0, sz)],
                             outh.at[j, pl.ds(off, sz)], outsems[st])
            for j in range(4)
        ]

    for ci in range(min(NUM_SETS, NUM_CHUNKS)):
        start_inputs(ci)
    for ci in range(NUM_CHUNKS):
        st = ci % NUM_SETS
        for h in in_handles[st]:
            h.wait()
        if ci >= NUM_SETS:
            for h in out_handles[st]:
                h.wait()
        _sort_chunk(in_bufs[st], out_bufs[st], SIZES[ci])
        nxt = ci + NUM_SETS
        if nxt < NUM_CHUNKS:
            start_inputs(nxt)
        start_outputs(ci)
    for st in range(NUM_SETS):
        if out_handles[st] is not None:
            for h in out_handles[st]:
                h.wait()


def kernel(x1, x2, x3, x4):
    mesh = plsc.VectorSubcoreMesh(core_axis_name="c", subcore_axis_name="s")
    run = functools.partial(
        pl.kernel,
        mesh=mesh,
        out_type=jax.ShapeDtypeStruct((4, N), jnp.float32),
        scratch_types=(
            [pltpu.VMEM((CHUNK,), jnp.float32) for _ in range(NUM_SETS * 8)]
            + [pltpu.SemaphoreType.DMA for _ in range(NUM_SETS * 2)]
        ),
    )(_body)
    return run(x1, x2, x3, x4)


# speedup vs baseline: 1.0437x; 1.0033x over previous
import functools

import jax
import jax.numpy as jnp
from jax import lax
from jax.experimental import pallas as pl
from jax.experimental.pallas import tpu as pltpu
from jax.experimental.pallas import tpu_sc as plsc

N = 1048576
NUM_CORES = 2
NUM_SUBCORES = 16
NUM_WORKERS = NUM_CORES * NUM_SUBCORES
PER_WORKER = N // NUM_WORKERS
CHUNK = 4096
LANES = 16
NUM_SETS = 3

SIZES = (1024, 3072) + (CHUNK,) * 6 + (3072, 1024)
OFFSETS = tuple(sum(SIZES[:i]) for i in range(len(SIZES)))
NUM_CHUNKS = len(SIZES)
assert sum(SIZES) == PER_WORKER


def _sort_chunk(in_bufs, out_bufs, size):
    def inner(i, c):
        for u in range(2):
            s = i * (2 * LANES) + u * LANES
            a = in_bufs[0][pl.ds(s, LANES)]
            b = in_bufs[1][pl.ds(s, LANES)]
            cc = in_bufs[2][pl.ds(s, LANES)]
            d = in_bufs[3][pl.ds(s, LANES)]
            lo1 = jnp.minimum(a, b)
            hi1 = jnp.maximum(a, b)
            lo2 = jnp.minimum(cc, d)
            hi2 = jnp.maximum(cc, d)
            o0 = jnp.minimum(lo1, lo2)
            o3 = jnp.maximum(hi1, hi2)
            m1 = jnp.maximum(lo1, lo2)
            m2 = jnp.minimum(hi1, hi2)
            out_bufs[0][pl.ds(s, LANES)] = o0
            out_bufs[1][pl.ds(s, LANES)] = jnp.minimum(m1, m2)
            out_bufs[2][pl.ds(s, LANES)] = jnp.maximum(m1, m2)
            out_bufs[3][pl.ds(s, LANES)] = o3
        return c

    lax.fori_loop(0, size // LANES // 2, inner, 0)


def _body(x1h, x2h, x3h, x4h, outh, *scratch):
    nb = NUM_SETS * 4
    in_bufs = tuple(tuple(scratch[st * 4 + j] for j in range(4))
                    for st in range(NUM_SETS))
    out_bufs = tuple(tuple(scratch[nb + st * 4 + j] for j in range(4))
                     for st in range(NUM_SETS))
    insems = scratch[2 * nb:2 * nb + NUM_SETS]
    outsems = scratch[2 * nb + NUM_SETS:]
    wid = lax.axis_index("s") * NUM_CORES + lax.axis_index("c")
    base = wid * PER_WORKER
    xs = (x1h, x2h, x3h, x4h)

    in_handles = [None] * NUM_SETS
    out_handles = [None] * NUM_SETS

    def start_inputs(ci):
        st = ci % NUM_SETS
        off = base + OFFSETS[ci]
        sz = SIZES[ci]
        in_handles[st] = [
            pltpu.async_copy(xs[j].at[pl.ds(off, sz)],
                             in_bufs[st][j].at[pl.ds(0, sz)], insems[st])
            for j in range(4)
        ]

    def start_outputs(ci):
        st = ci % NUM_SETS
        off = base + OFFSETS[ci]
        sz = SIZES[ci]
        out_handles[st] = [
            pltpu.async_copy(out_bufs[st][j].at[pl.ds(0, sz)],
                             outh.at[j, pl.ds(off, sz)], outsems[st])
            for j in range(4)
        ]

    for ci in range(min(NUM_SETS, NUM_CHUNKS)):
        start_inputs(ci)
    for ci in range(NUM_CHUNKS):
        st = ci % NUM_SETS
        for h in in_handles[st]:
            h.wait()
        if ci >= NUM_SETS:
            for h in out_handles[st]:
                h.wait()
        _sort_chunk(in_bufs[st], out_bufs[st], SIZES[ci])
        nxt = ci + NUM_SETS
        if nxt < NUM_CHUNKS:
            start_inputs(nxt)
        start_outputs(ci)
    for st in range(NUM_SETS):
        if out_handles[st] is not None:
            for h in out_handles[st]:
                h.wait()


def kernel(x1, x2, x3, x4):
    mesh = plsc.VectorSubcoreMesh(core_axis_name="c", subcore_axis_name="s")
    run = functools.partial(
        pl.kernel,
        mesh=mesh,
        out_type=jax.ShapeDtypeStruct((4, N), jnp.float32),
        scratch_types=(
            [pltpu.VMEM((CHUNK,), jnp.float32) for _ in range(NUM_SETS * 8)]
            + [pltpu.SemaphoreType.DMA for _ in range(NUM_SETS * 2)]
        ),
    )(_body)
    return run(x1, x2, x3, x4)
